# Initial kernel scaffold; baseline (speedup 1.0000x reference)
import jax
import jax.numpy as jnp
from jax.experimental import pallas as pl


def kernel(inputs, edge_index, W0, al0, ar0, b0, W1, al1, ar1, b1, W2, al2, ar2, b2):
    raise NotImplementedError("write your pallas kernel here")


# speedup vs baseline: 48.0018x; 48.0018x over previous
import functools

import jax
import jax.numpy as jnp
from jax import lax
from jax.experimental import pallas as pl
from jax.experimental.pallas import tpu as pltpu
from jax.experimental.pallas import tpu_sc as plsc

N = 10000
E = 320000
NC = 2
NS = 16
NW = NC * NS
EPW = E // NW
K = 80
NCHUNK = EPW // K
RPT = N // NS
RZB = 125


def _sc_edge_pass(H, FW):
  D = 16
  EL0 = H * D
  mesh = plsc.VectorSubcoreMesh(core_axis_name="c", subcore_axis_name="s")

  @functools.partial(
      pl.kernel,
      out_type=jax.ShapeDtypeStruct((NC, N, FW), jnp.float32),
      mesh=mesh,
      compiler_params=pltpu.CompilerParams(use_tc_tiling_on_sc=False),
      scratch_types=[
          pltpu.VMEM_SHARED((N, FW), jnp.float32),
          pltpu.VMEM((K,), jnp.int32),
          pltpu.VMEM((K,), jnp.int32),
          pltpu.VMEM((K, FW), jnp.float32),
          pltpu.VMEM((K, 16), jnp.float32),
          pltpu.VMEM((16,), jnp.float32),
          pltpu.VMEM((RZB, FW), jnp.float32),
          pltpu.SemaphoreType.DMA,
          pltpu.SemaphoreType.DMA,
      ],
  )
  def edge_kernel(featx, er16, srcs, dsts, cvec, out, acc, sidx, didx,
                  rows, erb, cbuf, zbuf, sem1, sem2):
    cid = lax.axis_index("c")
    sid = lax.axis_index("s")
    wid = sid * NC + cid

    zero = jnp.zeros((16,), jnp.float32)

    def zrow(r, carry):
      for j in range(FW // 16):
        zbuf[r, pl.ds(j * 16, 16)] = zero
      return carry

    lax.fori_loop(0, RZB, zrow, 0)
    for j in range(RPT // RZB):
      pltpu.sync_copy(zbuf, acc.at[pl.ds(sid * RPT + j * RZB, RZB)])
    pltpu.sync_copy(cvec, cbuf)
    plsc.subcore_barrier()

    cv = cbuf[...]
    mask = lax.iota(jnp.int32, 16) < H

    def chunk(i, carry):
      off = pl.multiple_of(wid * EPW + i * K, 8)
      pltpu.sync_copy(srcs.at[pl.ds(off, K)], sidx)
      pltpu.sync_copy(dsts.at[pl.ds(off, K)], didx)
      pltpu.async_copy(featx.at[sidx], rows, sem1).wait()
      pltpu.async_copy(er16.at[didx], erb, sem2).wait()

      def edge(e, ecarry):
        elv = rows[e, pl.ds(EL0, 16)]
        erv = erb[e]
        t = elv + erv
        t = jnp.maximum(t, 0.2 * t)
        ee = jnp.exp(t - cv)
        for h in range(H):
          idx = jnp.full((16,), h, jnp.int32)
          bc = jnp.take_along_axis(ee, idx, axis=0, mode="promise_in_bounds")
          rows[e, pl.ds(h * D, 16)] = rows[e, pl.ds(h * D, 16)] * bc
        rows[e, pl.ds(EL0, 16)] = jnp.where(mask, ee, 0.0)
        return ecarry

      lax.fori_loop(0, K, edge, 0)
      pltpu.sync_copy(rows, acc.at[didx], add=True)
      return carry

    lax.fori_loop(0, NCHUNK, chunk, 0)
    plsc.subcore_barrier()
    r0 = sid * RPT
    pltpu.sync_copy(acc.at[pl.ds(r0, RPT)], out.at[cid, pl.ds(r0, RPT)])

  return edge_kernel


_sc_l01 = _sc_edge_pass(8, 144)
_sc_l2 = _sc_edge_pass(1, 32)

BN = 2000
GRID = N // BN


def _full(shape):
  return pl.BlockSpec(shape, lambda i: (0,) * len(shape))


def _blk(w):
  return pl.BlockSpec((BN, w), lambda i: (i, 0))


def _tc_pre(x, W, Ablk, Bblk, FW, H):
  HD = H * 16
  pad = FW - HD - H

  def body(x_ref, w_ref, a_ref, b_ref, fx_ref, el_ref, er_ref):
    feat = jnp.dot(x_ref[...], w_ref[...], preferred_element_type=jnp.float32)
    el = jnp.dot(feat, a_ref[...], preferred_element_type=jnp.float32)
    er = jnp.dot(feat, b_ref[...], preferred_element_type=jnp.float32)
    z = jnp.zeros((BN, pad), jnp.float32)
    fx_ref[...] = jnp.concatenate([feat, el, z], axis=1)
    el_ref[...] = el
    er_ref[...] = jnp.concatenate(
        [er, jnp.zeros((BN, 16 - H), jnp.float32)], axis=1)

  return pl.pallas_call(
      body,
      grid=(GRID,),
      in_specs=[_blk(x.shape[1]), _full(W.shape), _full(Ablk.shape),
                _full(Bblk.shape)],
      out_specs=[_blk(FW), _blk(H), _blk(16)],
      out_shape=[
          jax.ShapeDtypeStruct((N, FW), jnp.float32),
          jax.ShapeDtypeStruct((N, H), jnp.float32),
          jax.ShapeDtypeStruct((N, 16), jnp.float32),
      ],
  )(x, W, Ablk, Bblk)


def _tc_mid(p0, p1, R, bvec, W, Ablk, Bblk, FW, H):
  HD = H * 16
  pad = FW - HD - H

  def body(p0_ref, p1_ref, r_ref, b_ref, w_ref, a_ref, bb_ref,
           fx_ref, el_ref, er_ref):
    acc = p0_ref[...] + p1_ref[...]
    esum = acc[:, 128:136]
    recip = jnp.where(esum > 0, 1.0 / esum, 0.0)
    scale = jnp.dot(recip, r_ref[...], preferred_element_type=jnp.float32)
    h = jnp.maximum(acc[:, 0:128] * scale + b_ref[...], 0.0)
    feat = jnp.dot(h, w_ref[...], preferred_element_type=jnp.float32)
    el = jnp.dot(feat, a_ref[...], preferred_element_type=jnp.float32)
    er = jnp.dot(feat, bb_ref[...], preferred_element_type=jnp.float32)
    z = jnp.zeros((BN, pad), jnp.float32)
    fx_ref[...] = jnp.concatenate([feat, el, z], axis=1)
    el_ref[...] = el
    er_ref[...] = jnp.concatenate(
        [er, jnp.zeros((BN, 16 - H), jnp.float32)], axis=1)

  return pl.pallas_call(
      body,
      grid=(GRID,),
      in_specs=[_blk(144), _blk(144), _full(R.shape), _full(bvec.shape),
                _full(W.shape), _full(Ablk.shape), _full(Bblk.shape)],
      out_specs=[_blk(FW), _blk(H), _blk(16)],
      out_shape=[
          jax.ShapeDtypeStruct((N, FW), jnp.float32),
          jax.ShapeDtypeStruct((N, H), jnp.float32),
          jax.ShapeDtypeStruct((N, 16), jnp.float32),
      ],
  )(p0, p1, R, bvec, W, Ablk, Bblk)


def _tc_final(q0, q1, b2vec):
  def body(q0_ref, q1_ref, b_ref, o_ref):
    acc = q0_ref[...] + q1_ref[...]
    esum = acc[:, 16:17]
    recip = jnp.where(esum > 0, 1.0 / esum, 0.0)
    o_ref[...] = acc[:, 0:16] * recip + b_ref[...]

  return pl.pallas_call(
      body,
      grid=(GRID,),
      in_specs=[_blk(32), _blk(32), _full(b2vec.shape)],
      out_specs=_blk(16),
      out_shape=jax.ShapeDtypeStruct((N, 16), jnp.float32),
  )(q0, q1, b2vec)


def _head_proj(a):
  H = a.shape[0]
  return (a[:, :, None] * jnp.eye(H, dtype=a.dtype)[:, None, :]).reshape(
      H * 16, H)


def _cvec(el, er16, H):
  c = jnp.maximum(jnp.max(el, axis=0) + jnp.max(er16[:, :H], axis=0), 0.0)
  return jnp.tile(c, 16 // H)


@jax.jit
def _run(inputs, edge_index, W0, al0, ar0, b0, W1, al1, ar1, b1,
         W2, al2, ar2, b2):
  src = edge_index[0]
  dst = edge_index[1]
  R = jnp.repeat(jnp.eye(8, dtype=jnp.float32), 16, axis=1)

  fx0, el0, er0 = _tc_pre(inputs, W0, _head_proj(al0), _head_proj(ar0), 144, 8)
  p = _sc_l01(fx0, er0, src, dst, _cvec(el0, er0, 8))

  fx1, el1, er1 = _tc_mid(p[0], p[1], R, b0.reshape(1, 128), W1,
                          _head_proj(al1), _head_proj(ar1), 144, 8)
  p = _sc_l01(fx1, er1, src, dst, _cvec(el1, er1, 8))

  fx2, el2, er2 = _tc_mid(p[0], p[1], R, b1.reshape(1, 128), W2,
                          _head_proj(al2), _head_proj(ar2), 32, 1)
  q = _sc_l2(fx2, er2, src, dst, _cvec(el2, er2, 1))

  return _tc_final(q[0], q[1], b2.reshape(1, 16))


def kernel(inputs, edge_index, W0, al0, ar0, b0, W1, al1, ar1, b1,
           W2, al2, ar2, b2):
  return _run(inputs, edge_index, W0, al0, ar0, b0, W1, al1, ar1, b1,
              W2, al2, ar2, b2)


# speedup vs baseline: 70.4718x; 1.4681x over previous
import functools

import jax
import jax.numpy as jnp
from jax import lax
from jax.experimental import pallas as pl
from jax.experimental.pallas import tpu as pltpu
from jax.experimental.pallas import tpu_sc as plsc

N = 10000
E = 320000
NC = 2
NS = 16
NW = NC * NS
EPW = E // NW
K = 80
NCHUNK = EPW // K
RPT = N // NS
RZB = 25


def _sc_edge_pass(H, FW):
  D = 16
  EL0 = H * D
  mesh = plsc.VectorSubcoreMesh(core_axis_name="c", subcore_axis_name="s")

  @functools.partial(
      pl.kernel,
      out_type=jax.ShapeDtypeStruct((NC, N, FW), jnp.float32),
      mesh=mesh,
      compiler_params=pltpu.CompilerParams(use_tc_tiling_on_sc=False),
      scratch_types=[
          pltpu.VMEM_SHARED((N, FW), jnp.float32),
          pltpu.VMEM((2, K), jnp.int32),
          pltpu.VMEM((2, K), jnp.int32),
          pltpu.VMEM((2, K, FW), jnp.float32),
          pltpu.VMEM((2, K, 16), jnp.float32),
          pltpu.VMEM((16,), jnp.float32),
          pltpu.VMEM((RZB, FW), jnp.float32),
          pltpu.SemaphoreType.DMA,
          pltpu.SemaphoreType.DMA,
          pltpu.SemaphoreType.DMA,
          pltpu.SemaphoreType.DMA,
      ],
  )
  def edge_kernel(featx, er16, srcs, dsts, cvec, out, acc, sidx, didx,
                  rows, erb, cbuf, zbuf, semr0, semr1, seme0, seme1):
    cid = lax.axis_index("c")
    sid = lax.axis_index("s")
    wid = sid * NC + cid
    semr = (semr0, semr1)
    seme = (seme0, seme1)

    zero = jnp.zeros((16,), jnp.float32)

    def zrow(r, carry):
      for j in range(FW // 16):
        zbuf[r, pl.ds(j * 16, 16)] = zero
      return carry

    lax.fori_loop(0, RZB, zrow, 0)
    for j in range(RPT // RZB):
      pltpu.sync_copy(zbuf, acc.at[pl.ds(sid * RPT + j * RZB, RZB)])
    pltpu.sync_copy(cvec, cbuf)
    plsc.subcore_barrier()

    cv = cbuf[...]
    mask = lax.iota(jnp.int32, 16) < H

    def start_gather(ci, b):
      off = pl.multiple_of(wid * EPW + ci * K, 8)
      pltpu.sync_copy(srcs.at[pl.ds(off, K)], sidx.at[b])
      pltpu.sync_copy(dsts.at[pl.ds(off, K)], didx.at[b])
      pltpu.make_async_copy(featx.at[sidx.at[b]], rows.at[b], semr[b]).start()
      pltpu.make_async_copy(er16.at[didx.at[b]], erb.at[b], seme[b]).start()

    def wait_gather(b):
      pltpu.make_async_copy(featx.at[sidx.at[b]], rows.at[b], semr[b]).wait()
      pltpu.make_async_copy(er16.at[didx.at[b]], erb.at[b], seme[b]).wait()

    def process(b):
      rb = rows.at[b]
      eb = erb.at[b]

      def edge(e):
        elv = rb[e, pl.ds(EL0, 16)]
        erv = eb[e]
        t = elv + erv
        t = jnp.maximum(t, 0.2 * t)
        ee = jnp.exp(t - cv)
        for h in range(H):
          idx = jnp.full((16,), h, jnp.int32)
          bc = jnp.take_along_axis(ee, idx, axis=0, mode="promise_in_bounds")
          rb[e, pl.ds(h * D, 16)] = rb[e, pl.ds(h * D, 16)] * bc
        rb[e, pl.ds(EL0, 16)] = jnp.where(mask, ee, 0.0)

      lax.fori_loop(0, K, lambda e, c: (edge(e), c)[1], 0)
      pltpu.sync_copy(rb, acc.at[didx.at[b]], add=True)

    start_gather(0, 0)

    def pair(it, carry):
      c0 = it * 2
      start_gather(c0 + 1, 1)
      wait_gather(0)
      process(0)
      start_gather(c0 + 2, 0)
      wait_gather(1)
      process(1)
      return carry

    lax.fori_loop(0, (NCHUNK - 1) // 2, pair, 0)
    wait_gather(0)
    process(0)
    plsc.subcore_barrier()
    r0 = sid * RPT
    pltpu.sync_copy(acc.at[pl.ds(r0, RPT)], out.at[cid, pl.ds(r0, RPT)])

  return edge_kernel


_sc_l01 = _sc_edge_pass(8, 144)
_sc_l2 = _sc_edge_pass(1, 32)

BN = 2000
GRID = N // BN


def _full(shape):
  return pl.BlockSpec(shape, lambda i: (0,) * len(shape))


def _blk(w):
  return pl.BlockSpec((BN, w), lambda i: (i, 0))


def _tc_pre(x, W, Ablk, Bblk, FW, H):
  HD = H * 16
  pad = FW - HD - H

  def body(x_ref, w_ref, a_ref, b_ref, fx_ref, el_ref, er_ref):
    feat = jnp.dot(x_ref[...], w_ref[...], preferred_element_type=jnp.float32)
    el = jnp.dot(feat, a_ref[...], preferred_element_type=jnp.float32)
    er = jnp.dot(feat, b_ref[...], preferred_element_type=jnp.float32)
    z = jnp.zeros((BN, pad), jnp.float32)
    fx_ref[...] = jnp.concatenate([feat, el, z], axis=1)
    el_ref[...] = el
    er_ref[...] = jnp.concatenate(
        [er, jnp.zeros((BN, 16 - H), jnp.float32)], axis=1)

  return pl.pallas_call(
      body,
      grid=(GRID,),
      in_specs=[_blk(x.shape[1]), _full(W.shape), _full(Ablk.shape),
                _full(Bblk.shape)],
      out_specs=[_blk(FW), _blk(H), _blk(16)],
      out_shape=[
          jax.ShapeDtypeStruct((N, FW), jnp.float32),
          jax.ShapeDtypeStruct((N, H), jnp.float32),
          jax.ShapeDtypeStruct((N, 16), jnp.float32),
      ],
  )(x, W, Ablk, Bblk)


def _tc_mid(p0, p1, R, bvec, W, Ablk, Bblk, FW, H):
  HD = H * 16
  pad = FW - HD - H

  def body(p0_ref, p1_ref, r_ref, b_ref, w_ref, a_ref, bb_ref,
           fx_ref, el_ref, er_ref):
    acc = p0_ref[...] + p1_ref[...]
    esum = acc[:, 128:136]
    recip = jnp.where(esum > 0, 1.0 / esum, 0.0)
    scale = jnp.dot(recip, r_ref[...], preferred_element_type=jnp.float32)
    h = jnp.maximum(acc[:, 0:128] * scale + b_ref[...], 0.0)
    feat = jnp.dot(h, w_ref[...], preferred_element_type=jnp.float32)
    el = jnp.dot(feat, a_ref[...], preferred_element_type=jnp.float32)
    er = jnp.dot(feat, bb_ref[...], preferred_element_type=jnp.float32)
    z = jnp.zeros((BN, pad), jnp.float32)
    fx_ref[...] = jnp.concatenate([feat, el, z], axis=1)
    el_ref[...] = el
    er_ref[...] = jnp.concatenate(
        [er, jnp.zeros((BN, 16 - H), jnp.float32)], axis=1)

  return pl.pallas_call(
      body,
      grid=(GRID,),
      in_specs=[_blk(144), _blk(144), _full(R.shape), _full(bvec.shape),
                _full(W.shape), _full(Ablk.shape), _full(Bblk.shape)],
      out_specs=[_blk(FW), _blk(H), _blk(16)],
      out_shape=[
          jax.ShapeDtypeStruct((N, FW), jnp.float32),
          jax.ShapeDtypeStruct((N, H), jnp.float32),
          jax.ShapeDtypeStruct((N, 16), jnp.float32),
      ],
  )(p0, p1, R, bvec, W, Ablk, Bblk)


def _tc_final(q0, q1, b2vec):
  def body(q0_ref, q1_ref, b_ref, o_ref):
    acc = q0_ref[...] + q1_ref[...]
    esum = acc[:, 16:17]
    recip = jnp.where(esum > 0, 1.0 / esum, 0.0)
    o_ref[...] = acc[:, 0:16] * recip + b_ref[...]

  return pl.pallas_call(
      body,
      grid=(GRID,),
      in_specs=[_blk(32), _blk(32), _full(b2vec.shape)],
      out_specs=_blk(16),
      out_shape=jax.ShapeDtypeStruct((N, 16), jnp.float32),
  )(q0, q1, b2vec)


def _head_proj(a):
  H = a.shape[0]
  return (a[:, :, None] * jnp.eye(H, dtype=a.dtype)[:, None, :]).reshape(
      H * 16, H)


def _cvec(el, er16, H):
  c = jnp.maximum(jnp.max(el, axis=0) + jnp.max(er16[:, :H], axis=0), 0.0)
  return jnp.tile(c, 16 // H)


@jax.jit
def _run(inputs, edge_index, W0, al0, ar0, b0, W1, al1, ar1, b1,
         W2, al2, ar2, b2):
  src = edge_index[0]
  dst = edge_index[1]
  R = jnp.repeat(jnp.eye(8, dtype=jnp.float32), 16, axis=1)

  fx0, el0, er0 = _tc_pre(inputs, W0, _head_proj(al0), _head_proj(ar0), 144, 8)
  p = _sc_l01(fx0, er0, src, dst, _cvec(el0, er0, 8))

  fx1, el1, er1 = _tc_mid(p[0], p[1], R, b0.reshape(1, 128), W1,
                          _head_proj(al1), _head_proj(ar1), 144, 8)
  p = _sc_l01(fx1, er1, src, dst, _cvec(el1, er1, 8))

  fx2, el2, er2 = _tc_mid(p[0], p[1], R, b1.reshape(1, 128), W2,
                          _head_proj(al2), _head_proj(ar2), 32, 1)
  q = _sc_l2(fx2, er2, src, dst, _cvec(el2, er2, 1))

  return _tc_final(q[0], q[1], b2.reshape(1, 16))


def kernel(inputs, edge_index, W0, al0, ar0, b0, W1, al1, ar1, b1,
           W2, al2, ar2, b2):
  return _run(inputs, edge_index, W0, al0, ar0, b0, W1, al1, ar1, b1,
              W2, al2, ar2, b2)
